# Initial kernel scaffold; baseline (speedup 1.0000x reference)
import jax
import jax.numpy as jnp
from jax.experimental import pallas as pl


def kernel(x, in_degree, out_degree, z_in, z_out):
    raise NotImplementedError("write your pallas kernel here")


# speedup vs baseline: 2.4415x; 2.4415x over previous
import functools

import jax
import jax.numpy as jnp
from jax import lax
from jax.experimental import pallas as pl
from jax.experimental.pallas import tpu as pltpu
from jax.experimental.pallas import tpu_sc as plsc

N = 100000
D = 128
C = 128
NC = 2
NS = 16
NW = NC * NS
N_CHUNKS = (N + C - 1) // C
CHUNKS_PER_W = (N_CHUNKS + NW - 1) // NW
LAST_BASE = N - C
LANES = 16
VECS_PER_ROW = D // LANES


def _sc_kernel(x_hbm, din_hbm, dout_hbm, zin_hbm, zout_hbm, out_hbm,
               idx_in, idx_out, xbuf, zinbuf, zoutbuf, sem_in, sem_out):
    cid = lax.axis_index("c")
    sid = lax.axis_index("s")
    wid = sid * NC + cid

    def chunk_body(j, carry):
        c = wid + j * NW

        @pl.when(c < N_CHUNKS)
        def _():
            base = jnp.minimum(c * C, LAST_BASE)
            pltpu.sync_copy(din_hbm.at[pl.ds(base, C)], idx_in)
            pltpu.sync_copy(dout_hbm.at[pl.ds(base, C)], idx_out)
            gin = pltpu.async_copy(zin_hbm.at[idx_in], zinbuf, sem_in)
            gout = pltpu.async_copy(zout_hbm.at[idx_out], zoutbuf, sem_out)
            pltpu.sync_copy(x_hbm.at[pl.ds(base, C)], xbuf)
            gin.wait()
            gout.wait()

            def row_body(r, carry2):
                for k in range(VECS_PER_ROW):
                    sl = pl.ds(k * LANES, LANES)
                    v = xbuf[r, sl] + zinbuf[r, sl] + zoutbuf[r, sl]
                    xbuf[r, sl] = v
                return carry2

            lax.fori_loop(0, C, row_body, 0)
            pltpu.sync_copy(xbuf, out_hbm.at[pl.ds(base, C)])

        return carry

    lax.fori_loop(0, CHUNKS_PER_W, chunk_body, 0)


@jax.jit
def _run(x, in_degree, out_degree, z_in, z_out):
    mesh = plsc.VectorSubcoreMesh(core_axis_name="c", subcore_axis_name="s")
    kern = functools.partial(
        pl.kernel,
        mesh=mesh,
        out_type=jax.ShapeDtypeStruct((N, D), jnp.float32),
        scratch_types=[
            pltpu.VMEM((C,), jnp.int32),
            pltpu.VMEM((C,), jnp.int32),
            pltpu.VMEM((C, D), jnp.float32),
            pltpu.VMEM((C, D), jnp.float32),
            pltpu.VMEM((C, D), jnp.float32),
            pltpu.SemaphoreType.DMA,
            pltpu.SemaphoreType.DMA,
        ],
    )(_sc_kernel)
    return kern(x, in_degree, out_degree, z_in, z_out)


def kernel(x, in_degree, out_degree, z_in, z_out):
    return _run(x, in_degree.astype(jnp.int32), out_degree.astype(jnp.int32),
                z_in, z_out)


# speedup vs baseline: 2.4783x; 1.0151x over previous
import functools

import jax
import jax.numpy as jnp
from jax import lax
from jax.experimental import pallas as pl
from jax.experimental.pallas import tpu as pltpu
from jax.experimental.pallas import tpu_sc as plsc

N = 100000
D = 128
C = 128
NC = 2
NS = 16
NW = NC * NS
N_CHUNKS = (N + C - 1) // C
CHUNKS_PER_W = (N_CHUNKS + NW - 1) // NW
LAST_BASE = N - C
LANES = 16
VECS_PER_ROW = D // LANES


def _sc_kernel(x_hbm, din_hbm, dout_hbm, zin_hbm, zout_hbm, out_hbm,
               idx_in, idx_out, xbuf, zinbuf, zoutbuf, sem_in, sem_out):
    cid = lax.axis_index("c")
    sid = lax.axis_index("s")
    wid = sid * NC + cid

    def chunk_body(j, carry):
        c = wid + j * NW

        @pl.when(c < N_CHUNKS)
        def _():
            base = jnp.minimum(c * C, LAST_BASE)
            pltpu.sync_copy(din_hbm.at[pl.ds(base, C)], idx_in)
            pltpu.sync_copy(dout_hbm.at[pl.ds(base, C)], idx_out)
            pltpu.sync_copy(x_hbm.at[pl.ds(base, C)], xbuf)
            pltpu.async_copy(zin_hbm.at[idx_in], xbuf, sem_in, add=True).wait()
            pltpu.async_copy(zout_hbm.at[idx_out], xbuf, sem_out, add=True).wait()
            pltpu.sync_copy(xbuf, out_hbm.at[pl.ds(base, C)])

        return carry

    lax.fori_loop(0, CHUNKS_PER_W, chunk_body, 0)


@jax.jit
def _run(x, in_degree, out_degree, z_in, z_out):
    mesh = plsc.VectorSubcoreMesh(core_axis_name="c", subcore_axis_name="s")
    kern = functools.partial(
        pl.kernel,
        mesh=mesh,
        out_type=jax.ShapeDtypeStruct((N, D), jnp.float32),
        scratch_types=[
            pltpu.VMEM((C,), jnp.int32),
            pltpu.VMEM((C,), jnp.int32),
            pltpu.VMEM((C, D), jnp.float32),
            pltpu.VMEM((C, D), jnp.float32),
            pltpu.VMEM((C, D), jnp.float32),
            pltpu.SemaphoreType.DMA,
            pltpu.SemaphoreType.DMA,
        ],
    )(_sc_kernel)
    return kern(x, in_degree, out_degree, z_in, z_out)


def kernel(x, in_degree, out_degree, z_in, z_out):
    return _run(x, in_degree.astype(jnp.int32), out_degree.astype(jnp.int32),
                z_in, z_out)


# speedup vs baseline: 3.3232x; 1.3409x over previous
import functools

import jax
import jax.numpy as jnp
from jax import lax
from jax.experimental import pallas as pl
from jax.experimental.pallas import tpu as pltpu
from jax.experimental.pallas import tpu_sc as plsc

N = 100000
D = 128
C = 128
NC = 2
NS = 16
NW = NC * NS
N_CHUNKS = (N + C - 1) // C
CHUNKS_PER_W = (N_CHUNKS + NW - 1) // NW
LAST_BASE = N - C
LANES = 16
VECS_PER_ROW = D // LANES


def _sc_kernel(x_hbm, din_hbm, dout_hbm, zin_hbm, zout_hbm, out_hbm,
               idx_in, idx_out, xbuf, zin_sh, zout_sh, sem_in, sem_out):
    cid = lax.axis_index("c")
    sid = lax.axis_index("s")
    wid = sid * NC + cid

    @pl.when(sid == 0)
    def _():
        pltpu.sync_copy(zin_hbm, zin_sh)
        pltpu.sync_copy(zout_hbm, zout_sh)

    plsc.subcore_barrier()

    def chunk_body(j, carry):
        c = wid + j * NW

        @pl.when(c < N_CHUNKS)
        def _():
            base = jnp.minimum(c * C, LAST_BASE)
            pltpu.sync_copy(din_hbm.at[pl.ds(base, C)], idx_in)
            pltpu.sync_copy(dout_hbm.at[pl.ds(base, C)], idx_out)
            pltpu.sync_copy(x_hbm.at[pl.ds(base, C)], xbuf)
            pltpu.async_copy(zin_sh.at[idx_in], xbuf, sem_in, add=True).wait()
            pltpu.async_copy(zout_sh.at[idx_out], xbuf, sem_out, add=True).wait()
            pltpu.sync_copy(xbuf, out_hbm.at[pl.ds(base, C)])

        return carry

    lax.fori_loop(0, CHUNKS_PER_W, chunk_body, 0)


@jax.jit
def _run(x, in_degree, out_degree, z_in, z_out):
    mesh = plsc.VectorSubcoreMesh(core_axis_name="c", subcore_axis_name="s")
    kern = functools.partial(
        pl.kernel,
        mesh=mesh,
        out_type=jax.ShapeDtypeStruct((N, D), jnp.float32),
        scratch_types=[
            pltpu.VMEM((C,), jnp.int32),
            pltpu.VMEM((C,), jnp.int32),
            pltpu.VMEM((C, D), jnp.float32),
            pltpu.VMEM_SHARED((512, D), jnp.float32),
            pltpu.VMEM_SHARED((512, D), jnp.float32),
            pltpu.SemaphoreType.DMA,
            pltpu.SemaphoreType.DMA,
        ],
    )(_sc_kernel)
    return kern(x, in_degree, out_degree, z_in, z_out)


def kernel(x, in_degree, out_degree, z_in, z_out):
    return _run(x, in_degree.astype(jnp.int32), out_degree.astype(jnp.int32),
                z_in, z_out)


# speedup vs baseline: 6.1497x; 1.8506x over previous
import functools

import jax
import jax.numpy as jnp
from jax import lax
from jax.experimental import pallas as pl
from jax.experimental.pallas import tpu as pltpu
from jax.experimental.pallas import tpu_sc as plsc

N = 100000
D = 128
C = 128
NC = 2
NS = 16
NW = NC * NS
N_CHUNKS = (N + C - 1) // C
CHUNKS_PER_W = (N_CHUNKS + NW - 1) // NW
LAST_BASE = N - C
NBUF = 4
T_TOTAL = CHUNKS_PER_W + 3


def _sc_kernel(x_hbm, din_hbm, dout_hbm, zin_hbm, zout_hbm, out_hbm,
               idx_in, idx_out, xbuf, zin_sh, zout_sh, sem_i, sem_g, sem_o):
    cid = lax.axis_index("c")
    sid = lax.axis_index("s")
    wid = sid * NC + cid

    @pl.when(sid == 0)
    def _():
        pltpu.sync_copy(zin_hbm, zin_sh)
        pltpu.sync_copy(zout_hbm, zout_sh)

    plsc.subcore_barrier()

    def drain_in(b):
        pltpu.make_async_copy(din_hbm.at[pl.ds(0, C)], idx_in.at[b], sem_i.at[b]).wait()
        pltpu.make_async_copy(dout_hbm.at[pl.ds(0, C)], idx_out.at[b], sem_i.at[b]).wait()
        pltpu.make_async_copy(x_hbm.at[pl.ds(0, C)], xbuf.at[b], sem_i.at[b]).wait()

    def drain_g(b):
        pltpu.make_async_copy(x_hbm.at[pl.ds(0, C)], xbuf.at[b], sem_g.at[b]).wait()

    def drain_o(b):
        pltpu.make_async_copy(xbuf.at[b], out_hbm.at[pl.ds(0, C)], sem_o.at[b]).wait()

    def stage_a(t, b):
        c = wid + t * NW

        @pl.when(c < N_CHUNKS)
        def _():
            @pl.when(t >= NBUF)
            def _():
                drain_o(b)
            base = jnp.minimum(c * C, LAST_BASE)
            pltpu.async_copy(din_hbm.at[pl.ds(base, C)], idx_in.at[b], sem_i.at[b])
            pltpu.async_copy(dout_hbm.at[pl.ds(base, C)], idx_out.at[b], sem_i.at[b])
            pltpu.async_copy(x_hbm.at[pl.ds(base, C)], xbuf.at[b], sem_i.at[b])

    def stage_b(t, b):
        c = wid + t * NW

        @pl.when((t >= 0) & (c < N_CHUNKS))
        def _():
            drain_in(b)
            pltpu.async_copy(zin_sh.at[idx_in.at[b]], xbuf.at[b], sem_g.at[b], add=True)

    def stage_c(t, b):
        c = wid + t * NW

        @pl.when((t >= 0) & (c < N_CHUNKS))
        def _():
            drain_g(b)
            pltpu.async_copy(zout_sh.at[idx_out.at[b]], xbuf.at[b], sem_g.at[b], add=True)

    def stage_d(t, b):
        c = wid + t * NW

        @pl.when((t >= 0) & (c < N_CHUNKS))
        def _():
            drain_g(b)
            base = jnp.minimum(c * C, LAST_BASE)
            pltpu.async_copy(xbuf.at[b], out_hbm.at[pl.ds(base, C)], sem_o.at[b])

    def outer_body(t0, carry):
        t = t0 * NBUF
        for u in range(NBUF):
            stage_d(t + u - 3, (u + 1) % NBUF)
            stage_c(t + u - 2, (u + 2) % NBUF)
            stage_b(t + u - 1, (u + 3) % NBUF)
            stage_a(t + u, u)
        return carry

    lax.fori_loop(0, T_TOTAL // NBUF, outer_body, 0)

    for t in range(CHUNKS_PER_W - NBUF, CHUNKS_PER_W):
        b = t % NBUF

        @pl.when(wid + t * NW < N_CHUNKS)
        def _():
            drain_o(b)


@jax.jit
def _run(x, in_degree, out_degree, z_in, z_out):
    mesh = plsc.VectorSubcoreMesh(core_axis_name="c", subcore_axis_name="s")
    kern = functools.partial(
        pl.kernel,
        mesh=mesh,
        out_type=jax.ShapeDtypeStruct((N, D), jnp.float32),
        scratch_types=[
            pltpu.VMEM((NBUF, C), jnp.int32),
            pltpu.VMEM((NBUF, C), jnp.int32),
            pltpu.VMEM((NBUF, C, D), jnp.float32),
            pltpu.VMEM_SHARED((512, D), jnp.float32),
            pltpu.VMEM_SHARED((512, D), jnp.float32),
            pltpu.SemaphoreType.DMA((NBUF,)),
            pltpu.SemaphoreType.DMA((NBUF,)),
            pltpu.SemaphoreType.DMA((NBUF,)),
        ],
    )(_sc_kernel)
    return kern(x, in_degree, out_degree, z_in, z_out)


def kernel(x, in_degree, out_degree, z_in, z_out):
    return _run(x, in_degree.astype(jnp.int32), out_degree.astype(jnp.int32),
                z_in, z_out)
